# baseline (speedup 1.0000x reference)
import functools
import math

import jax
import jax.numpy as jnp
import numpy as np
from jax import lax
from jax.experimental import pallas as pl
from jax.experimental.pallas import tpu as pltpu
from jax.experimental.pallas import tpu_sc as plsc

B = 64
D_IN = 1024
H_PHASOR = 32
DELTA0 = 7.0
HIDDEN_DIM = 512
VOCAB_SIZE = 100000
NUM_EXPERTS = 8
EXPERT_DIM = 32
K_WINNERS = 5
DECAY = 0.7
THETA = 1.0
D_ENH = D_IN + 2 * H_PHASOR

VOCAB_TILE = 8192
NUM_TILES = (VOCAB_SIZE + VOCAB_TILE - 1) // VOCAB_TILE

LANES = 16
NCHUNK = HIDDEN_DIM // LANES
NEG = float(np.finfo(np.float32).min)

def _gain_const(r):
    v = np.float32(1.0)
    for _ in range(r):
        v = np.float32(v * np.float32(DECAY))
    return float(1.0 + 1.0 / (1.0 + math.exp(-(float(v) - THETA))))

GAINS = [_gain_const(r) for r in range(K_WINNERS)]


def _moe_context(b, x, gw, gb, w1_ref, b1_ref, w2_ref, b2_ref):
    xm = jnp.mean(x, axis=1, keepdims=True)
    h = (jax.lax.broadcasted_iota(jnp.int32, (1, H_PHASOR), 1)
         .astype(jnp.float32) + 1.0)
    phase = DELTA0 * xm * h
    xe = jnp.concatenate([x, jnp.cos(phase), jnp.sin(phase)], axis=1)

    logits = jnp.dot(xe, gw, preferred_element_type=jnp.float32) + gb
    iota_e = jax.lax.broadcasted_iota(jnp.int32, (b, NUM_EXPERTS), 1)
    m1 = jnp.max(logits, axis=1, keepdims=True)
    idx1 = jnp.min(jnp.where(logits == m1, iota_e, NUM_EXPERTS),
                   axis=1, keepdims=True)
    l2 = jnp.where(iota_e == idx1, -jnp.inf, logits)
    m2 = jnp.max(l2, axis=1, keepdims=True)
    idx2 = jnp.min(jnp.where(l2 == m2, iota_e, NUM_EXPERTS),
                   axis=1, keepdims=True)
    e2 = jnp.exp(m2 - m1)
    g1 = 1.0 / (1.0 + e2)
    g2 = e2 * g1
    wgt = (jnp.where(iota_e == idx1, g1, 0.0)
           + jnp.where(iota_e == idx2, g2, 0.0))

    ctx = jnp.zeros((b, HIDDEN_DIM), jnp.float32)
    for e in range(NUM_EXPERTS):
        he = jnp.maximum(
            jnp.dot(xe, w1_ref[e], preferred_element_type=jnp.float32)
            + b1_ref[e:e + 1, :], 0.0)
        oe = (jnp.dot(he, w2_ref[e], preferred_element_type=jnp.float32)
              + b2_ref[e:e + 1, :])
        ctx = ctx + wgt[:, e:e + 1] * oe
    return ctx


def _ctx0_body(x0_ref, gw_ref, gb_ref, w1_ref, b1_ref, w2_ref, b2_ref,
               ctx0_ref):
    ctx0_ref[...] = _moe_context(1, x0_ref[...], gw_ref[...], gb_ref[...],
                                 w1_ref, b1_ref, w2_ref, b2_ref)


def _ctx0(x0, gate_W, gb2, exp_W1, exp_b1, exp_W2, exp_b2):
    return pl.pallas_call(
        _ctx0_body,
        out_shape=jax.ShapeDtypeStruct((1, HIDDEN_DIM), jnp.float32),
    )(x0, gate_W, gb2, exp_W1, exp_b1, exp_W2, exp_b2)


def _bcast_max(v):
    return plsc.cummax(jnp.flip(plsc.cummax(v), 0))


def _gains_body(ctx_hbm, gvec_hbm, row_v, gv_v):
    cid = lax.axis_index("c")
    sid = lax.axis_index("s")

    @pl.when((cid == 0) & (sid == 0))
    def _():
        pltpu.sync_copy(ctx_hbm.at[0], row_v)
        lanes = lax.iota(jnp.int32, LANES)
        regs = [row_v[pl.ds(LANES * c, LANES)] for c in range(NCHUNK)]
        poss = []
        for r in range(K_WINNERS):
            bv = regs[0]
            bi = lanes
            for c in range(1, NCHUNK):
                v = regs[c]
                idx = LANES * c + lanes
                better = v > bv
                bv = jnp.where(better, v, bv)
                bi = jnp.where(better, idx, bi)
            m = _bcast_max(bv)
            cand = jnp.where(bv == m, bi, HIDDEN_DIM)
            pos = -_bcast_max(-cand)
            poss.append(pos)
            regs = [jnp.where((LANES * c + lanes) == pos, NEG, regs[c])
                    for c in range(NCHUNK)]
        ones = jnp.ones((LANES,), jnp.float32)
        for c in range(NCHUNK):
            idx = LANES * c + lanes
            g = ones
            for r in range(K_WINNERS):
                g = jnp.where(idx == poss[r], GAINS[r], g)
            gv_v[pl.ds(LANES * c, LANES)] = g
        pltpu.sync_copy(gv_v, gvec_hbm)


_gains = functools.partial(
    pl.kernel,
    out_type=jax.ShapeDtypeStruct((HIDDEN_DIM,), jnp.float32),
    mesh=plsc.VectorSubcoreMesh(core_axis_name="c", subcore_axis_name="s"),
    scratch_types=[
        pltpu.VMEM((HIDDEN_DIM,), jnp.float32),
        pltpu.VMEM((HIDDEN_DIM,), jnp.float32),
    ],
    compiler_params=pltpu.CompilerParams(needs_layout_passes=False),
)(_gains_body)


def _ctx_full_body(x_ref, gw_ref, gb_ref, w1_ref, b1_ref, w2_ref, b2_ref,
                   ctx_ref):
    ctx_ref[...] = _moe_context(B, x_ref[...], gw_ref[...], gb_ref[...],
                                w1_ref, b1_ref, w2_ref, b2_ref)


def _ctx_full(x, gate_W, gb2, exp_W1, exp_b1, exp_W2, exp_b2):
    return pl.pallas_call(
        _ctx_full_body,
        out_shape=jax.ShapeDtypeStruct((B, HIDDEN_DIM), jnp.float32),
    )(x, gate_W, gb2, exp_W1, exp_b1, exp_W2, exp_b2)


def _matmul_body(ctx_ref, gv_ref, wout_ref, bout_ref, out_ref):
    att = ctx_ref[...] * gv_ref[...]
    out_ref[...] = (jnp.dot(att, wout_ref[...],
                            preferred_element_type=jnp.float32)
                    + bout_ref[...])


def _matmul(ctx, gv2, W_out, bout2):
    return pl.pallas_call(
        _matmul_body,
        grid=(NUM_TILES,),
        in_specs=[
            pl.BlockSpec((B, HIDDEN_DIM), lambda i: (0, 0)),
            pl.BlockSpec((1, HIDDEN_DIM), lambda i: (0, 0)),
            pl.BlockSpec((HIDDEN_DIM, VOCAB_TILE), lambda i: (0, i)),
            pl.BlockSpec((1, VOCAB_TILE), lambda i: (0, i)),
        ],
        out_specs=pl.BlockSpec((B, VOCAB_TILE), lambda i: (0, i)),
        out_shape=jax.ShapeDtypeStruct((B, VOCAB_SIZE), jnp.float32),
        compiler_params=pltpu.CompilerParams(
            dimension_semantics=("arbitrary",)),
    )(ctx, gv2, W_out, bout2)


def kernel(x, gate_W, gate_b, exp_W1, exp_b1, exp_W2, exp_b2, W_out, b_out):
    gb2 = gate_b.reshape(1, NUM_EXPERTS)
    bout2 = b_out.reshape(1, VOCAB_SIZE)
    ctx0 = _ctx0(x[0:1], gate_W, gb2, exp_W1, exp_b1, exp_W2, exp_b2)
    gvec = _gains(ctx0)
    ctx = _ctx_full(x, gate_W, gb2, exp_W1, exp_b1, exp_W2, exp_b2)
    return _matmul(ctx, gvec.reshape(1, HIDDEN_DIM), W_out, bout2)


# speedup vs baseline: 1.0175x; 1.0175x over previous
import functools
import math

import jax
import jax.numpy as jnp
import numpy as np
from jax import lax
from jax.experimental import pallas as pl
from jax.experimental.pallas import tpu as pltpu
from jax.experimental.pallas import tpu_sc as plsc

B = 64
D_IN = 1024
H_PHASOR = 32
DELTA0 = 7.0
HIDDEN_DIM = 512
VOCAB_SIZE = 100000
NUM_EXPERTS = 8
EXPERT_DIM = 32
K_WINNERS = 5
DECAY = 0.7
THETA = 1.0
D_ENH = D_IN + 2 * H_PHASOR

VOCAB_TILE = 8192
NUM_TILES = (VOCAB_SIZE + VOCAB_TILE - 1) // VOCAB_TILE

LANES = 16
NCHUNK = HIDDEN_DIM // LANES
NEG = float(np.finfo(np.float32).min)

def _gain_const(r):
    v = np.float32(1.0)
    for _ in range(r):
        v = np.float32(v * np.float32(DECAY))
    return float(1.0 + 1.0 / (1.0 + math.exp(-(float(v) - THETA))))

GAINS = [_gain_const(r) for r in range(K_WINNERS)]


def _moe_context(b, x, gw, gb, w1_ref, b1_ref, w2_ref, b2_ref):
    xm = jnp.mean(x, axis=1, keepdims=True)
    h = (jax.lax.broadcasted_iota(jnp.int32, (1, H_PHASOR), 1)
         .astype(jnp.float32) + 1.0)
    phase = DELTA0 * xm * h
    xe = jnp.concatenate([x, jnp.cos(phase), jnp.sin(phase)], axis=1)

    logits = jnp.dot(xe, gw, preferred_element_type=jnp.float32) + gb
    iota_e = jax.lax.broadcasted_iota(jnp.int32, (b, NUM_EXPERTS), 1)
    m1 = jnp.max(logits, axis=1, keepdims=True)
    idx1 = jnp.min(jnp.where(logits == m1, iota_e, NUM_EXPERTS),
                   axis=1, keepdims=True)
    l2 = jnp.where(iota_e == idx1, -jnp.inf, logits)
    m2 = jnp.max(l2, axis=1, keepdims=True)
    idx2 = jnp.min(jnp.where(l2 == m2, iota_e, NUM_EXPERTS),
                   axis=1, keepdims=True)
    e2 = jnp.exp(m2 - m1)
    g1 = 1.0 / (1.0 + e2)
    g2 = e2 * g1
    wgt = (jnp.where(iota_e == idx1, g1, 0.0)
           + jnp.where(iota_e == idx2, g2, 0.0))

    ctx = jnp.zeros((b, HIDDEN_DIM), jnp.float32)
    for e in range(NUM_EXPERTS):
        he = jnp.maximum(
            jnp.dot(xe, w1_ref[e], preferred_element_type=jnp.float32)
            + b1_ref[e:e + 1, :], 0.0)
        oe = (jnp.dot(he, w2_ref[e], preferred_element_type=jnp.float32)
              + b2_ref[e:e + 1, :])
        ctx = ctx + wgt[:, e:e + 1] * oe
    return ctx


def _bcast_max(v):
    return plsc.cummax(jnp.flip(plsc.cummax(v), 0))


def _gains_body(ctx_hbm, gvec_hbm, row_v, gv_v):
    cid = lax.axis_index("c")
    sid = lax.axis_index("s")

    @pl.when((cid == 0) & (sid == 0))
    def _():
        pltpu.sync_copy(ctx_hbm.at[0], row_v)
        lanes = lax.iota(jnp.int32, LANES)
        regs = [row_v[pl.ds(LANES * c, LANES)] for c in range(NCHUNK)]
        poss = []
        for r in range(K_WINNERS):
            bv = regs[0]
            bi = lanes
            for c in range(1, NCHUNK):
                v = regs[c]
                idx = LANES * c + lanes
                better = v > bv
                bv = jnp.where(better, v, bv)
                bi = jnp.where(better, idx, bi)
            m = _bcast_max(bv)
            cand = jnp.where(bv == m, bi, HIDDEN_DIM)
            pos = -_bcast_max(-cand)
            poss.append(pos)
            regs = [jnp.where((LANES * c + lanes) == pos, NEG, regs[c])
                    for c in range(NCHUNK)]
        ones = jnp.ones((LANES,), jnp.float32)
        for c in range(NCHUNK):
            idx = LANES * c + lanes
            g = ones
            for r in range(K_WINNERS):
                g = jnp.where(idx == poss[r], GAINS[r], g)
            gv_v[pl.ds(LANES * c, LANES)] = g
        pltpu.sync_copy(gv_v, gvec_hbm)


_gains = functools.partial(
    pl.kernel,
    out_type=jax.ShapeDtypeStruct((HIDDEN_DIM,), jnp.float32),
    mesh=plsc.VectorSubcoreMesh(core_axis_name="c", subcore_axis_name="s"),
    scratch_types=[
        pltpu.VMEM((HIDDEN_DIM,), jnp.float32),
        pltpu.VMEM((HIDDEN_DIM,), jnp.float32),
    ],
    compiler_params=pltpu.CompilerParams(needs_layout_passes=False),
)(_gains_body)


def _ctx_full_body(x_ref, gw_ref, gb_ref, w1_ref, b1_ref, w2_ref, b2_ref,
                   ctx_ref):
    ctx_ref[...] = _moe_context(B, x_ref[...], gw_ref[...], gb_ref[...],
                                w1_ref, b1_ref, w2_ref, b2_ref)


def _ctx_full(x, gate_W, gb2, exp_W1, exp_b1, exp_W2, exp_b2):
    return pl.pallas_call(
        _ctx_full_body,
        out_shape=jax.ShapeDtypeStruct((B, HIDDEN_DIM), jnp.float32),
    )(x, gate_W, gb2, exp_W1, exp_b1, exp_W2, exp_b2)


def _matmul_body(ctx_ref, gv_ref, wout_ref, bout_ref, out_ref):
    att = ctx_ref[...] * gv_ref[...]
    out_ref[...] = (jnp.dot(att, wout_ref[...],
                            preferred_element_type=jnp.float32)
                    + bout_ref[...])


def _matmul(ctx, gv2, W_out, bout2):
    return pl.pallas_call(
        _matmul_body,
        grid=(NUM_TILES,),
        in_specs=[
            pl.BlockSpec((B, HIDDEN_DIM), lambda i: (0, 0)),
            pl.BlockSpec((1, HIDDEN_DIM), lambda i: (0, 0)),
            pl.BlockSpec((HIDDEN_DIM, VOCAB_TILE), lambda i: (0, i)),
            pl.BlockSpec((1, VOCAB_TILE), lambda i: (0, i)),
        ],
        out_specs=pl.BlockSpec((B, VOCAB_TILE), lambda i: (0, i)),
        out_shape=jax.ShapeDtypeStruct((B, VOCAB_SIZE), jnp.float32),
        compiler_params=pltpu.CompilerParams(
            dimension_semantics=("arbitrary",)),
    )(ctx, gv2, W_out, bout2)


def kernel(x, gate_W, gate_b, exp_W1, exp_b1, exp_W2, exp_b2, W_out, b_out):
    gb2 = gate_b.reshape(1, NUM_EXPERTS)
    bout2 = b_out.reshape(1, VOCAB_SIZE)
    ctx = _ctx_full(x, gate_W, gb2, exp_W1, exp_b1, exp_W2, exp_b2)
    gvec = _gains(ctx)
    return _matmul(ctx, gvec.reshape(1, HIDDEN_DIM), W_out, bout2)


# speedup vs baseline: 1.0257x; 1.0081x over previous
import functools
import math

import jax
import jax.numpy as jnp
import numpy as np
from jax import lax
from jax.experimental import pallas as pl
from jax.experimental.pallas import tpu as pltpu
from jax.experimental.pallas import tpu_sc as plsc

B = 64
D_IN = 1024
H_PHASOR = 32
DELTA0 = 7.0
HIDDEN_DIM = 512
VOCAB_SIZE = 100000
NUM_EXPERTS = 8
EXPERT_DIM = 32
K_WINNERS = 5
DECAY = 0.7
THETA = 1.0
D_ENH = D_IN + 2 * H_PHASOR

VOCAB_TILE = 8192
NUM_TILES = (VOCAB_SIZE + VOCAB_TILE - 1) // VOCAB_TILE

LANES = 16
NCHUNK = HIDDEN_DIM // LANES
NEG = float(np.finfo(np.float32).min)

def _gain_const(r):
    v = np.float32(1.0)
    for _ in range(r):
        v = np.float32(v * np.float32(DECAY))
    return float(1.0 + 1.0 / (1.0 + math.exp(-(float(v) - THETA))))

GAINS = [_gain_const(r) for r in range(K_WINNERS)]


def _moe_context(b, x, gw, gb, w1_ref, b1_ref, w2_ref, b2_ref):
    xm = jnp.mean(x, axis=1, keepdims=True)
    h = (jax.lax.broadcasted_iota(jnp.int32, (1, H_PHASOR), 1)
         .astype(jnp.float32) + 1.0)
    phase = DELTA0 * xm * h
    xe = jnp.concatenate([x, jnp.cos(phase), jnp.sin(phase)], axis=1)

    logits = jnp.dot(xe, gw, preferred_element_type=jnp.float32) + gb
    iota_e = jax.lax.broadcasted_iota(jnp.int32, (b, NUM_EXPERTS), 1)
    m1 = jnp.max(logits, axis=1, keepdims=True)
    idx1 = jnp.min(jnp.where(logits == m1, iota_e, NUM_EXPERTS),
                   axis=1, keepdims=True)
    l2 = jnp.where(iota_e == idx1, -jnp.inf, logits)
    m2 = jnp.max(l2, axis=1, keepdims=True)
    idx2 = jnp.min(jnp.where(l2 == m2, iota_e, NUM_EXPERTS),
                   axis=1, keepdims=True)
    e2 = jnp.exp(m2 - m1)
    g1 = 1.0 / (1.0 + e2)
    g2 = e2 * g1
    wgt = (jnp.where(iota_e == idx1, g1, 0.0)
           + jnp.where(iota_e == idx2, g2, 0.0))

    ctx = jnp.zeros((b, HIDDEN_DIM), jnp.float32)
    for e in range(NUM_EXPERTS):
        he = jnp.maximum(
            jnp.dot(xe, w1_ref[e], preferred_element_type=jnp.float32)
            + b1_ref[e:e + 1, :], 0.0)
        oe = (jnp.dot(he, w2_ref[e], preferred_element_type=jnp.float32)
              + b2_ref[e:e + 1, :])
        ctx = ctx + wgt[:, e:e + 1] * oe
    return ctx


def _bcast_max(v):
    return plsc.cummax(jnp.flip(plsc.cummax(v), 0))


def _gains_body(ctx_hbm, gvec_hbm, row_v, gv_v):
    cid = lax.axis_index("c")
    sid = lax.axis_index("s")

    @pl.when((cid == 0) & (sid == 0))
    def _():
        pltpu.sync_copy(ctx_hbm.at[0], row_v)
        lanes = lax.iota(jnp.int32, LANES)
        regs = [row_v[pl.ds(LANES * c, LANES)] for c in range(NCHUNK)]
        poss = []
        for r in range(K_WINNERS):
            bv = regs[0]
            bi = lanes
            for c in range(1, NCHUNK):
                v = regs[c]
                idx = LANES * c + lanes
                better = v > bv
                bv = jnp.where(better, v, bv)
                bi = jnp.where(better, idx, bi)
            m = _bcast_max(bv)
            cand = jnp.where(bv == m, bi, HIDDEN_DIM)
            pos = -_bcast_max(-cand)
            poss.append(pos)
            regs = [jnp.where((LANES * c + lanes) == pos, NEG, regs[c])
                    for c in range(NCHUNK)]
        ones = jnp.ones((LANES,), jnp.float32)
        for c in range(NCHUNK):
            idx = LANES * c + lanes
            g = ones
            for r in range(K_WINNERS):
                g = jnp.where(idx == poss[r], GAINS[r], g)
            gv_v[pl.ds(LANES * c, LANES)] = g
        pltpu.sync_copy(gv_v, gvec_hbm)


_gains = functools.partial(
    pl.kernel,
    out_type=jax.ShapeDtypeStruct((HIDDEN_DIM,), jnp.float32),
    mesh=plsc.VectorSubcoreMesh(core_axis_name="c", subcore_axis_name="s", num_cores=1),
    scratch_types=[
        pltpu.VMEM((HIDDEN_DIM,), jnp.float32),
        pltpu.VMEM((HIDDEN_DIM,), jnp.float32),
    ],
    compiler_params=pltpu.CompilerParams(needs_layout_passes=False),
)(_gains_body)


def _ctx_full_body(x_ref, gw_ref, gb_ref, w1_ref, b1_ref, w2_ref, b2_ref,
                   ctx_ref):
    ctx_ref[...] = _moe_context(B, x_ref[...], gw_ref[...], gb_ref[...],
                                w1_ref, b1_ref, w2_ref, b2_ref)


def _ctx_full(x, gate_W, gb2, exp_W1, exp_b1, exp_W2, exp_b2):
    return pl.pallas_call(
        _ctx_full_body,
        out_shape=jax.ShapeDtypeStruct((B, HIDDEN_DIM), jnp.float32),
    )(x, gate_W, gb2, exp_W1, exp_b1, exp_W2, exp_b2)


def _matmul_body(ctx_ref, gv_ref, wout_ref, bout_ref, out_ref):
    att = ctx_ref[...] * gv_ref[...]
    out_ref[...] = (jnp.dot(att, wout_ref[...],
                            preferred_element_type=jnp.float32)
                    + bout_ref[...])


def _matmul(ctx, gv2, W_out, bout2):
    return pl.pallas_call(
        _matmul_body,
        grid=(NUM_TILES,),
        in_specs=[
            pl.BlockSpec((B, HIDDEN_DIM), lambda i: (0, 0)),
            pl.BlockSpec((1, HIDDEN_DIM), lambda i: (0, 0)),
            pl.BlockSpec((HIDDEN_DIM, VOCAB_TILE), lambda i: (0, i)),
            pl.BlockSpec((1, VOCAB_TILE), lambda i: (0, i)),
        ],
        out_specs=pl.BlockSpec((B, VOCAB_TILE), lambda i: (0, i)),
        out_shape=jax.ShapeDtypeStruct((B, VOCAB_SIZE), jnp.float32),
        compiler_params=pltpu.CompilerParams(
            dimension_semantics=("arbitrary",)),
    )(ctx, gv2, W_out, bout2)


def kernel(x, gate_W, gate_b, exp_W1, exp_b1, exp_W2, exp_b2, W_out, b_out):
    gb2 = gate_b.reshape(1, NUM_EXPERTS)
    bout2 = b_out.reshape(1, VOCAB_SIZE)
    ctx = _ctx_full(x, gate_W, gb2, exp_W1, exp_b1, exp_W2, exp_b2)
    gvec = _gains(ctx)
    return _matmul(ctx, gvec.reshape(1, HIDDEN_DIM), W_out, bout2)
